# baseline (speedup 1.0000x reference)
import functools

import jax
import jax.numpy as jnp
from jax import lax
from jax.experimental import pallas as pl
from jax.experimental.pallas import tpu as pltpu, tpu_sc as plsc

SEQ = 2048
ROWLEN = SEQ + 2
B = 4096
NC, NS, L = 2, 16, 16
NW = NC * NS
ROWS_PER_W = B // NW
BLK = 8
NBLK = ROWS_PER_W // BLK
NCHUNK = SEQ // L


def _body(in_hbm, out1_hbm, out2_hbm, blockbuf, m1buf, m2buf, o1buf, o2buf):
    wid = lax.axis_index("s") * NC + lax.axis_index("c")
    base = wid * ROWS_PER_W
    lane_i = lax.iota(jnp.int32, L)
    lane_f = lane_i.astype(jnp.float32)
    ones = jnp.ones((L,), jnp.float32)

    def do_block(blk, carry_none):
        rowstart = base + blk * BLK
        pltpu.sync_copy(in_hbm.at[pl.ds(rowstart * ROWLEN, BLK * ROWLEN)], blockbuf)

        rmp_vec = ones
        for r in range(BLK):
            rbase = r * ROWLEN
            mp_f = plsc.load_gather(blockbuf, [jnp.full((L,), rbase + SEQ, jnp.int32)])
            bid_f = plsc.load_gather(blockbuf, [jnp.full((L,), rbase + SEQ + 1, jnp.int32)])
            rmp = plsc.load_gather(blockbuf, [mp_f.astype(jnp.int32) + rbase])

            def chunk(j, carry):
                acc1, acc2, pos_f = carry
                v = blockbuf[pl.ds(rbase + j * L, L)]
                acc1 = acc1 * jnp.where(pos_f < bid_f, v, ones)
                acc2 = acc2 * jnp.where(pos_f < mp_f, v, ones)
                return acc1, acc2, pos_f + 16.0

            acc1, acc2, _ = lax.fori_loop(0, NCHUNK, chunk, (ones, ones, lane_f))

            col = lane_i * 17 + r
            plsc.store_scatter(m1buf, [col], acc1)
            plsc.store_scatter(m2buf, [col], acc2)
            rmp_vec = jnp.where(lane_i == r, rmp, rmp_vec)

        p1 = m1buf[pl.ds(0, L)]
        p2 = m2buf[pl.ds(0, L)]
        for l in range(1, L):
            p1 = p1 * m1buf[pl.ds(l * 17, L)]
            p2 = p2 * m2buf[pl.ds(l * 17, L)]

        o1buf[pl.ds(blk * BLK, L)] = p1
        o2buf[pl.ds(blk * BLK, L)] = p2 * (ones - rmp_vec)
        return carry_none

    lax.fori_loop(0, NBLK, do_block, None)
    pltpu.sync_copy(o1buf.at[pl.ds(0, ROWS_PER_W)], out1_hbm.at[pl.ds(base, ROWS_PER_W)])
    pltpu.sync_copy(o2buf.at[pl.ds(0, ROWS_PER_W)], out2_hbm.at[pl.ds(base, ROWS_PER_W)])


@jax.jit
def kernel(inputs):
    mesh = plsc.VectorSubcoreMesh(core_axis_name="c", subcore_axis_name="s")
    f = pl.kernel(
        _body,
        out_type=(
            jax.ShapeDtypeStruct((B,), jnp.float32),
            jax.ShapeDtypeStruct((B,), jnp.float32),
        ),
        mesh=mesh,
        compiler_params=pltpu.CompilerParams(use_tc_tiling_on_sc=False, needs_layout_passes=False),
        scratch_types=[
            pltpu.VMEM((BLK * ROWLEN,), jnp.float32),
            pltpu.VMEM((L * 17,), jnp.float32),
            pltpu.VMEM((L * 17,), jnp.float32),
            pltpu.VMEM((ROWS_PER_W + L - BLK,), jnp.float32),
            pltpu.VMEM((ROWS_PER_W + L - BLK,), jnp.float32),
        ],
    )
    out1, out2 = f(inputs.reshape(B * ROWLEN))
    return out1.reshape(B, 1), out2.reshape(B, 1)


# speedup vs baseline: 1.3988x; 1.3988x over previous
import jax
import jax.numpy as jnp
from jax import lax
from jax.experimental import pallas as pl
from jax.experimental.pallas import tpu as pltpu, tpu_sc as plsc

SEQ = 2048
ROWLEN = SEQ + 2
B = 4096
NC, NS, L = 2, 16, 16
NW = NC * NS
ROWS_PER_W = B // NW
BLK = 16
NBLK = ROWS_PER_W // BLK
CHW = BLK * ROWLEN


def _body(in_hbm, out1_hbm, out2_hbm, bufa, bufb, m1buf, m2buf, o1buf, o2buf, sema, semb):
    wid = lax.axis_index("s") * NC + lax.axis_index("c")
    base = wid * ROWS_PER_W
    lane_i = lax.iota(jnp.int32, L)
    ones = jnp.ones((L,), jnp.float32)

    def off(blk):
        return (base + blk * BLK) * ROWLEN

    def issue(blk, buf, sem):
        pltpu.async_copy(in_hbm.at[pl.ds(off(blk), CHW)], buf, sem)

    def wait(blk, buf, sem):
        pltpu.make_async_copy(in_hbm.at[pl.ds(off(blk), CHW)], buf, sem).wait()

    def process(buf, blk):
        def rowbody(r, rmp_vec):
            rbase = r * ROWLEN
            tv = buf[pl.ds(rbase + SEQ - 14, L)]
            mp_s = tv[14].astype(jnp.int32)
            bid_s = tv[15].astype(jnp.int32)
            lo = jnp.minimum(mp_s, bid_s)
            hi = jnp.maximum(mp_s, bid_s)
            n_a = lo >> 4
            rem_a = lo & 15
            n_b = hi >> 4
            rem_b = hi & 15

            def ch2(j, c):
                a1, a2 = c
                p = rbase + j * 32
                return a1 * buf[pl.ds(p, L)], a2 * buf[pl.ds(p + 16, L)]

            a1, a2 = lax.fori_loop(0, n_a >> 1, ch2, (ones, ones))
            odd_a = lax.broadcast((n_a & 1) == 1, (L,))
            v_odd = buf[pl.ds(rbase + jnp.maximum(n_a - 1, 0) * 16, L)]
            acc_a = a1 * a2 * jnp.where(odd_a, v_odd, ones)

            v_a = buf[pl.ds(rbase + n_a * 16, L)]
            p_lo = acc_a * jnp.where(lane_i < lax.broadcast(rem_a, (L,)), v_a, ones)

            def chb(j, c):
                b1, b2 = c
                p = rbase + n_a * 16 + j * 32
                return b1 * buf[pl.ds(p, L)], b2 * buf[pl.ds(p + 16, L)]

            b1, b2 = lax.fori_loop(0, (n_b - n_a) >> 1, chb, (ones, ones))
            odd_b = lax.broadcast(((n_b - n_a) & 1) == 1, (L,))
            v_oddb = buf[pl.ds(rbase + jnp.maximum(n_b - 1, 0) * 16, L)]
            acc_b = acc_a * b1 * b2 * jnp.where(odd_b, v_oddb, ones)

            v_b = buf[pl.ds(rbase + n_b * 16, L)]
            p_hi = acc_b * jnp.where(lane_i < lax.broadcast(rem_b, (L,)), v_b, ones)

            rmp = plsc.load_gather(buf, [lax.broadcast(mp_s, (L,)) + rbase])

            cond = lax.broadcast(bid_s <= mp_s, (L,))
            p_bid = jnp.where(cond, p_lo, p_hi)
            p_mp = jnp.where(cond, p_hi, p_lo)

            col = lane_i * 17 + r
            plsc.store_scatter(m1buf, [col], p_bid)
            plsc.store_scatter(m2buf, [col], p_mp)
            return jnp.where(lane_i == r, rmp, rmp_vec)

        rmp_vec = lax.fori_loop(0, BLK, rowbody, ones)

        p1 = m1buf[pl.ds(0, L)]
        p2 = m2buf[pl.ds(0, L)]
        for l in range(1, L):
            p1 = p1 * m1buf[pl.ds(l * 17, L)]
            p2 = p2 * m2buf[pl.ds(l * 17, L)]
        o1buf[pl.ds(blk * BLK, L)] = p1
        o2buf[pl.ds(blk * BLK, L)] = p2 * (ones - rmp_vec)

    issue(0, bufa, sema)

    def pairbody(i, carry_none):
        blk_a = 2 * i
        blk_b = 2 * i + 1
        wait(blk_a, bufa, sema)
        issue(blk_b, bufb, semb)
        process(bufa, blk_a)
        wait(blk_b, bufb, semb)

        @pl.when(i < NBLK // 2 - 1)
        def _():
            issue(blk_a + 2, bufa, sema)

        process(bufb, blk_b)
        return carry_none

    lax.fori_loop(0, NBLK // 2, pairbody, None)
    pltpu.sync_copy(o1buf.at[pl.ds(0, ROWS_PER_W)], out1_hbm.at[pl.ds(base, ROWS_PER_W)])
    pltpu.sync_copy(o2buf.at[pl.ds(0, ROWS_PER_W)], out2_hbm.at[pl.ds(base, ROWS_PER_W)])


@jax.jit
def kernel(inputs):
    mesh = plsc.VectorSubcoreMesh(core_axis_name="c", subcore_axis_name="s")
    f = pl.kernel(
        _body,
        out_type=(
            jax.ShapeDtypeStruct((B,), jnp.float32),
            jax.ShapeDtypeStruct((B,), jnp.float32),
        ),
        mesh=mesh,
        compiler_params=pltpu.CompilerParams(use_tc_tiling_on_sc=False, needs_layout_passes=False),
        scratch_types=[
            pltpu.VMEM((CHW,), jnp.float32),
            pltpu.VMEM((CHW,), jnp.float32),
            pltpu.VMEM((L * 17,), jnp.float32),
            pltpu.VMEM((L * 17,), jnp.float32),
            pltpu.VMEM((ROWS_PER_W,), jnp.float32),
            pltpu.VMEM((ROWS_PER_W,), jnp.float32),
            pltpu.SemaphoreType.DMA,
            pltpu.SemaphoreType.DMA,
        ],
    )
    out1, out2 = f(inputs.reshape(B * ROWLEN))
    return out1.reshape(B, 1), out2.reshape(B, 1)


# speedup vs baseline: 4.9463x; 3.5360x over previous
import jax
import jax.numpy as jnp
from jax import lax
from jax.experimental import pallas as pl
from jax.experimental.pallas import tpu as pltpu, tpu_sc as plsc

SEQ = 2048
ROWLEN = SEQ + 2
B = 4096
NC, NS, L = 2, 16, 16
NW = NC * NS
BCOLS = B // NW
NG = BCOLS // L
CS = 256
NCHK = SEQ // CS
COCT = CS // 8
NST = 5


def _body(x_hbm, out1_hbm, out2_hbm, bufa, bufb, thbuf, stbuf, o1buf, o2buf, sema, semb):
    w = lax.axis_index("s") * NC + lax.axis_index("c")
    bcol0 = w * BCOLS
    lane_i = lax.iota(jnp.int32, L)
    ones = jnp.ones((L,), jnp.float32)
    zeros_i = jnp.zeros((L,), jnp.int32)

    pltpu.sync_copy(x_hbm.at[pl.ds(SEQ, 2), pl.ds(bcol0, BCOLS)], thbuf)

    def issue(c, buf, sem):
        pltpu.async_copy(x_hbm.at[pl.ds(c * CS, CS), pl.ds(bcol0, BCOLS)], buf, sem)

    def wait(c, buf, sem):
        pltpu.make_async_copy(x_hbm.at[pl.ds(c * CS, CS), pl.ds(bcol0, BCOLS)], buf, sem).wait()

    for k in range(NG * NST):
        stbuf[pl.ds(k * L, L)] = ones

    def group_precomp(gl):
        colbase = gl * L + lane_i
        mp = thbuf[0, pl.ds(gl * L, L)].astype(jnp.int32)
        bid = thbuf[1, pl.ds(gl * L, L)].astype(jnp.int32)
        return colbase, mp, bid

    def process(buf, c):
        c_v = jnp.full((L,), 0, jnp.int32) + c

        def groupbody(gl, carry_none):
            colbase, mp, bid = group_precomp(gl)
            g1 = lax.shift_right_logical(bid, 3)
            r1 = lax.bitwise_and(bid, 7)
            g2 = lax.shift_right_logical(mp, 3)
            r2 = lax.bitwise_and(mp, 7)
            sbase = gl * NST * L
            acc1 = stbuf[pl.ds(sbase, L)]
            acc2 = stbuf[pl.ds(sbase + L, L)]

            def octet(o, accs):
                a1, a2 = accs
                s0 = o * 8
                v0 = buf[s0, pl.ds(gl * L, L)]
                v1 = buf[s0 + 1, pl.ds(gl * L, L)]
                v2 = buf[s0 + 2, pl.ds(gl * L, L)]
                v3 = buf[s0 + 3, pl.ds(gl * L, L)]
                v4 = buf[s0 + 4, pl.ds(gl * L, L)]
                v5 = buf[s0 + 5, pl.ds(gl * L, L)]
                v6 = buf[s0 + 6, pl.ds(gl * L, L)]
                v7 = buf[s0 + 7, pl.ds(gl * L, L)]
                op = ((v0 * v1) * (v2 * v3)) * ((v4 * v5) * (v6 * v7))
                go = c_v * COCT + o
                a1 = a1 * jnp.where(go < g1, op, ones)
                a2 = a2 * jnp.where(go < g2, op, ones)
                return a1, a2

            acc1, acc2 = lax.fori_loop(0, COCT, octet, (acc1, acc2))
            stbuf[pl.ds(sbase, L)] = acc1
            stbuf[pl.ds(sbase + L, L)] = acc2

            def boundary(gk, rk):
                inch = (lax.shift_right_logical(gk, 5) == c_v)
                srow = lax.bitwise_and(gk, 31) * 8
                srow = jnp.where(inch, srow, zeros_i)
                fch = ones
                for j in range(7):
                    val = plsc.load_gather(buf, [srow + j, colbase])
                    m = jnp.logical_and(inch, jnp.full((L,), j, jnp.int32) < rk)
                    fch = fch * jnp.where(m, val, ones)
                return fch

            f1 = stbuf[pl.ds(sbase + 2 * L, L)] * boundary(g1, r1)
            f2 = stbuf[pl.ds(sbase + 3 * L, L)] * boundary(g2, r2)
            stbuf[pl.ds(sbase + 2 * L, L)] = f1
            stbuf[pl.ds(sbase + 3 * L, L)] = f2

            inm = (lax.shift_right_logical(mp, 8) == c_v)
            mrow = jnp.where(inm, lax.bitwise_and(mp, CS - 1), zeros_i)
            mval = plsc.load_gather(buf, [mrow, colbase])
            rmp = stbuf[pl.ds(sbase + 4 * L, L)]
            stbuf[pl.ds(sbase + 4 * L, L)] = jnp.where(inm, mval, rmp)
            return carry_none

        lax.fori_loop(0, NG, groupbody, None)

    issue(0, bufa, sema)

    def pairbody(i, carry_none):
        ca = 2 * i
        cb = 2 * i + 1
        wait(ca, bufa, sema)
        issue(cb, bufb, semb)
        process(bufa, ca)
        wait(cb, bufb, semb)

        @pl.when(i < NCHK // 2 - 1)
        def _():
            issue(ca + 2, bufa, sema)

        process(bufb, cb)
        return carry_none

    lax.fori_loop(0, NCHK // 2, pairbody, None)

    def final(gl, carry_none):
        sbase = gl * NST * L
        p1 = stbuf[pl.ds(sbase, L)] * stbuf[pl.ds(sbase + 2 * L, L)]
        p2 = stbuf[pl.ds(sbase + L, L)] * stbuf[pl.ds(sbase + 3 * L, L)]
        rmp = stbuf[pl.ds(sbase + 4 * L, L)]
        o1buf[pl.ds(gl * L, L)] = p1
        o2buf[pl.ds(gl * L, L)] = p2 * (ones - rmp)
        return carry_none

    lax.fori_loop(0, NG, final, None)
    pltpu.sync_copy(o1buf.at[pl.ds(0, BCOLS)], out1_hbm.at[pl.ds(bcol0, BCOLS)])
    pltpu.sync_copy(o2buf.at[pl.ds(0, BCOLS)], out2_hbm.at[pl.ds(bcol0, BCOLS)])


@jax.jit
def kernel(inputs):
    mesh = plsc.VectorSubcoreMesh(core_axis_name="c", subcore_axis_name="s")
    f = pl.kernel(
        _body,
        out_type=(
            jax.ShapeDtypeStruct((B,), jnp.float32),
            jax.ShapeDtypeStruct((B,), jnp.float32),
        ),
        mesh=mesh,
        compiler_params=pltpu.CompilerParams(use_tc_tiling_on_sc=True, needs_layout_passes=False),
        scratch_types=[
            pltpu.VMEM((CS, BCOLS), jnp.float32),
            pltpu.VMEM((CS, BCOLS), jnp.float32),
            pltpu.VMEM((2, BCOLS), jnp.float32),
            pltpu.VMEM((NG * NST * L,), jnp.float32),
            pltpu.VMEM((BCOLS,), jnp.float32),
            pltpu.VMEM((BCOLS,), jnp.float32),
            pltpu.SemaphoreType.DMA,
            pltpu.SemaphoreType.DMA,
        ],
    )
    out1, out2 = f(inputs.T)
    return out1.reshape(B, 1), out2.reshape(B, 1)
